# baseline (speedup 1.0000x reference)
import functools

import jax
import jax.numpy as jnp
from jax import lax
from jax.experimental import pallas as pl
from jax.experimental.pallas import tpu as pltpu
from jax.experimental.pallas import tpu_sc as plsc


def _make_sc_gather(num_entities: int, entity_dim: int, batch: int):
    info = plsc.get_sparse_core_info()
    nw = info.num_cores * info.num_subcores
    assert batch % nw == 0
    b_per_w = batch // nw

    mesh = plsc.VectorSubcoreMesh(core_axis_name="c", subcore_axis_name="s")

    @functools.partial(
        pl.kernel,
        mesh=mesh,
        out_type=jax.ShapeDtypeStruct((batch, entity_dim), jnp.float32),
        scratch_types=[
            pltpu.VMEM((b_per_w,), jnp.int32),
            pltpu.VMEM((b_per_w, entity_dim), jnp.float32),
            pltpu.SemaphoreType.DMA,
        ],
        compiler_params=pltpu.CompilerParams(use_tc_tiling_on_sc=False),
    )
    def gather_kernel(table_hbm, idx_hbm, out_hbm, idx_v, rows_v, sem):
        wid = lax.axis_index("s") * info.num_cores + lax.axis_index("c")
        base = wid * b_per_w
        pltpu.sync_copy(idx_hbm.at[pl.ds(base, b_per_w)], idx_v)
        pltpu.async_copy(table_hbm.at[idx_v], rows_v, sem).wait()
        pltpu.sync_copy(rows_v, out_hbm.at[pl.ds(base, b_per_w)])

    return gather_kernel


def _project_body(g_ref, w_ref, b_ref, o_ref):
    o_ref[...] = (
        jnp.dot(g_ref[...], w_ref[...], preferred_element_type=jnp.float32)
        + b_ref[...]
    )


def kernel(x, entity_bank, W, b):
    batch = x.shape[0]
    num_entities, entity_dim = entity_bank.shape
    out_dim = W.shape[1]

    idx = x.reshape(batch).astype(jnp.int32)
    gathered = _make_sc_gather(num_entities, entity_dim, batch)(entity_bank, idx)

    return pl.pallas_call(
        _project_body,
        out_shape=jax.ShapeDtypeStruct((batch, out_dim), jnp.float32),
    )(gathered, W, b.reshape(1, out_dim))


# speedup vs baseline: 1.6486x; 1.6486x over previous
import functools

import jax
import jax.numpy as jnp
from jax import lax
from jax.experimental import pallas as pl
from jax.experimental.pallas import tpu as pltpu
from jax.experimental.pallas import tpu_sc as plsc


def _make_sc_gather(num_groups: int, entity_dim: int, batch: int):
    info = plsc.get_sparse_core_info()
    nw = info.num_cores * info.num_subcores
    assert batch % nw == 0
    b_per_w = batch // nw
    lanes = info.num_lanes

    mesh = plsc.VectorSubcoreMesh(core_axis_name="c", subcore_axis_name="s")

    @functools.partial(
        pl.kernel,
        mesh=mesh,
        out_type=jax.ShapeDtypeStruct((batch, 128), jnp.float32),
        scratch_types=[
            pltpu.VMEM((batch,), jnp.int32),
            pltpu.VMEM((b_per_w, 8, entity_dim), jnp.float32),
            pltpu.VMEM((b_per_w, 128), jnp.float32),
            pltpu.SemaphoreType.DMA,
        ],
        compiler_params=pltpu.CompilerParams(needs_layout_passes=False),
    )
    def gather_kernel(table_hbm, idx_hbm, out_hbm, idx_v, rows_v, out_v, sem):
        wid = lax.axis_index("s") * info.num_cores + lax.axis_index("c")
        base = wid * b_per_w
        pltpu.sync_copy(idx_hbm, idx_v)
        copies = []
        for t in range(b_per_w // lanes):
            gv = idx_v[pl.ds(base + t * lanes, lanes)] >> 3
            for jj in range(lanes):
                j = t * lanes + jj
                g = gv[jj]
                copies.append(
                    pltpu.async_copy(
                        table_hbm.at[pl.ds(g, 1)], rows_v.at[pl.ds(j, 1)], sem
                    )
                )
        for c in copies:
            c.wait()
        for t in range(b_per_w // lanes):
            sv = idx_v[pl.ds(base + t * lanes, lanes)] & 7
            jv = lax.iota(jnp.int32, lanes) + t * lanes
            for c in range(entity_dim):
                cv = jnp.full((lanes,), c, jnp.int32)
                vals = plsc.load_gather(rows_v, [jv, sv, cv])
                plsc.store_scatter(out_v, [jv, cv], vals)
        pltpu.sync_copy(out_v, out_hbm.at[pl.ds(base, b_per_w)])

    return gather_kernel


def _project_body(g_ref, w_ref, b_ref, o_ref):
    o_ref[...] = (
        jnp.dot(g_ref[:, :32], w_ref[...], preferred_element_type=jnp.float32)
        + b_ref[...]
    )


def kernel(x, entity_bank, W, b):
    batch = x.shape[0]
    num_entities, entity_dim = entity_bank.shape
    out_dim = W.shape[1]

    idx = x.reshape(batch).astype(jnp.int32)
    table3 = entity_bank.reshape(num_entities // 8, 8, entity_dim)
    gathered = _make_sc_gather(num_entities // 8, entity_dim, batch)(table3, idx)

    return pl.pallas_call(
        _project_body,
        out_shape=jax.ShapeDtypeStruct((batch, out_dim), jnp.float32),
    )(gathered, W, b.reshape(1, out_dim))
